# Initial kernel scaffold; baseline (speedup 1.0000x reference)
import jax
import jax.numpy as jnp
from jax.experimental import pallas as pl


def kernel(input_ids, image_features, embed_tokens):
    raise NotImplementedError("write your pallas kernel here")


# speedup vs baseline: 1.7669x; 1.7669x over previous
import jax
import jax.numpy as jnp
from jax import lax
from jax.experimental import pallas as pl
from jax.experimental.pallas import tpu as pltpu
from jax.experimental.pallas import tpu_sc as plsc

_NUM_WORKERS = 32
_CHUNK = 16


def _gather_rows(table, flat_ids):
    n_ids = flat_ids.shape[0]
    d = table.shape[1]
    b_per_w = n_ids // _NUM_WORKERS
    n_chunks = b_per_w // _CHUNK
    assert b_per_w * _NUM_WORKERS == n_ids and n_chunks * _CHUNK == b_per_w
    assert n_chunks % 2 == 0
    mesh = plsc.VectorSubcoreMesh(core_axis_name="core",
                                  subcore_axis_name="subcore")

    @pl.kernel(
        out_type=jax.ShapeDtypeStruct((n_ids, d), table.dtype),
        mesh=mesh,
        scratch_types=[
            pltpu.VMEM((b_per_w,), jnp.int32),
            pltpu.VMEM((_CHUNK, d), table.dtype),
            pltpu.VMEM((_CHUNK, d), table.dtype),
            pltpu.SemaphoreType.DMA,
            pltpu.SemaphoreType.DMA,
        ],
    )
    def gather_kernel(table_hbm, ids_hbm, out_hbm, idx_v, rows0, rows1,
                      sem0, sem1):
        wid = lax.axis_index("subcore") * 2 + lax.axis_index("core")
        base = wid * b_per_w
        pltpu.sync_copy(ids_hbm.at[pl.ds(base, b_per_w)], idx_v)

        def start(c, buf, sem):
            pltpu.async_copy(table_hbm.at[idx_v.at[pl.ds(c * _CHUNK, _CHUNK)]],
                             buf, sem)

        def wait(buf, sem):
            pltpu.make_async_copy(table_hbm.at[pl.ds(0, _CHUNK)], buf,
                                  sem).wait()

        def write_out(c, buf):
            pltpu.sync_copy(buf, out_hbm.at[pl.ds(base + c * _CHUNK, _CHUNK)])

        start(0, rows0, sem0)

        @pl.loop(0, n_chunks, step=2)
        def _(c):
            start(c + 1, rows1, sem1)
            wait(rows0, sem0)
            write_out(c, rows0)

            @pl.when(c + 2 < n_chunks)
            def _():
                start(c + 2, rows0, sem0)

            wait(rows1, sem1)
            write_out(c + 1, rows1)

    return gather_kernel(table, flat_ids)


def kernel(input_ids, image_features, embed_tokens):
    del image_features
    batch, seq = input_ids.shape
    flat_ids = input_ids.reshape(batch * seq).astype(jnp.int32)
    rows = _gather_rows(embed_tokens, flat_ids)
    return rows.reshape(batch, seq, embed_tokens.shape[1])
